# baseline (speedup 1.0000x reference)
import jax
import jax.numpy as jnp
from jax.experimental import pallas as pl
from jax.experimental.pallas import tpu as pltpu

_TOK_ID = 12345
_STRENGTH = 18.0

_SEQ_BLK = 1024


def _body(li_ref, x_ref, w_ref, o_ref):
    b = pl.program_id(0)
    nj = pl.num_programs(1)
    o_ref[...] = x_ref[...]
    li = li_ref[b]
    if nj == 1:
        start = 0
    else:
        start = pl.program_id(1) * _SEQ_BLK

    @pl.when((li >= start) & (li < start + _SEQ_BLK))
    def _():
        w = w_ref[_TOK_ID % 8, :]
        norm = jnp.sqrt(jnp.sum(w * w))
        d = (_STRENGTH / jnp.maximum(norm, 1e-12)) * w
        r = li - start
        o_ref[pl.ds(r, 1), :] = x_ref[pl.ds(r, 1), :] + d[None, :]


def kernel(x, token_ids, last_indices, lm_head_weight):
    del token_ids
    B, S, D = x.shape
    grid = (B, S // _SEQ_BLK)
    return pl.pallas_call(
        _body,
        grid=grid,
        in_specs=[
            pl.BlockSpec(memory_space=pltpu.SMEM),
            pl.BlockSpec((None, _SEQ_BLK, D), lambda b, j: (b, j, 0)),
            pl.BlockSpec((8, D), lambda b, j: (_TOK_ID // 8, 0)),
        ],
        out_specs=pl.BlockSpec((None, _SEQ_BLK, D), lambda b, j: (b, j, 0)),
        out_shape=jax.ShapeDtypeStruct((B, S, D), x.dtype),
        compiler_params=pltpu.CompilerParams(
            dimension_semantics=("parallel", "parallel"),
        ),
    )(last_indices, x, lm_head_weight)


# speedup vs baseline: 1.0057x; 1.0057x over previous
import jax
import jax.numpy as jnp
from jax.experimental import pallas as pl
from jax.experimental.pallas import tpu as pltpu

_TOK_ID = 12345
_STRENGTH = 18.0

_NBUF = 4


def _body(li_ref, x_hbm, w_hbm, o_hbm, vrow, vw, bufs, sem_in, sem_out,
          sem_g, sem_s):
    B = vrow.shape[0]
    wcp = pltpu.make_async_copy(w_hbm.at[pl.ds(_TOK_ID, 1), :], vw, sem_g)
    wcp.start()
    gathers = []
    for b in range(B):
        li = li_ref[b]
        g = pltpu.make_async_copy(
            x_hbm.at[b, pl.ds(li, 1), :], vrow.at[pl.ds(b, 1), :], sem_g)
        g.start()
        gathers.append(g)

    in_cps = [None] * B
    out_cps = [None] * B
    scatters = []

    def start_in(i):
        c = pltpu.make_async_copy(
            x_hbm.at[i], bufs.at[i % _NBUF], sem_in.at[i % _NBUF])
        c.start()
        in_cps[i] = c

    def start_out(i):
        c = pltpu.make_async_copy(
            bufs.at[i % _NBUF], o_hbm.at[i], sem_out.at[i % _NBUF])
        c.start()
        out_cps[i] = c

    def start_scatter(i):
        li = li_ref[i]
        s = pltpu.make_async_copy(
            vrow.at[pl.ds(i, 1), :], o_hbm.at[i, pl.ds(li, 1), :], sem_s)
        s.start()
        scatters.append(s)

    for i in range(_NBUF):
        start_in(i)

    wcp.wait()
    for g in gathers:
        g.wait()
    w = vw[0, :]
    norm = jnp.sqrt(jnp.sum(w * w))
    d = (_STRENGTH / jnp.maximum(norm, 1e-12)) * w
    vrow[...] = vrow[...] + d[None, :]

    for i in range(B):
        if i >= _NBUF:
            out_cps[i - _NBUF].wait()
            start_scatter(i - _NBUF)
            start_in(i)
        j = i - (_NBUF - 1)
        if j >= 0:
            in_cps[j].wait()
            start_out(j)
    for j in range(B - (_NBUF - 1), B):
        in_cps[j].wait()
        start_out(j)
    for j in range(B - _NBUF, B):
        out_cps[j].wait()
        start_scatter(j)
    for s in scatters:
        s.wait()


def kernel(x, token_ids, last_indices, lm_head_weight):
    del token_ids
    B, S, D = x.shape
    return pl.pallas_call(
        _body,
        in_specs=[
            pl.BlockSpec(memory_space=pltpu.SMEM),
            pl.BlockSpec(memory_space=pltpu.MemorySpace.HBM),
            pl.BlockSpec(memory_space=pltpu.MemorySpace.HBM),
        ],
        out_specs=pl.BlockSpec(memory_space=pltpu.MemorySpace.HBM),
        out_shape=jax.ShapeDtypeStruct((B, S, D), x.dtype),
        scratch_shapes=[
            pltpu.VMEM((B, D), jnp.float32),
            pltpu.VMEM((1, D), jnp.float32),
            pltpu.VMEM((_NBUF, S, D), jnp.float32),
            pltpu.SemaphoreType.DMA((_NBUF,)),
            pltpu.SemaphoreType.DMA((_NBUF,)),
            pltpu.SemaphoreType.DMA,
            pltpu.SemaphoreType.DMA,
        ],
    )(last_indices, x, lm_head_weight)
